# baseline (speedup 1.0000x reference)
import jax
import jax.numpy as jnp
from jax import lax
from jax.experimental import pallas as pl
from jax.experimental.pallas import tpu as pltpu
from jax.experimental.pallas import tpu_sc as plsc

B, C, H, W = 4, 19, 224, 224
N = H * W
NUNITS = B * C
NC, NS, L = 2, 16, 16
NW = NC * NS
NTAIL = NUNITS - 2 * NW
RPC = 32
CH = RPC * W
GPR = W // L
NCHU = H // RPC
NB = 1024


def _sc_body(p_hbm, lab_hbm, out_hbm, p_v, lab_v, hist_v, merge_v, res_v,
             shared, sem_p, sem_l):
    cid = lax.axis_index("c")
    sid = lax.axis_index("s")
    w = sid * NC + cid
    ones = jnp.ones((L,), jnp.int32)

    def zero_hist():
        @plsc.parallel_loop(0, 2 * NB // L, unroll=8)
        def _(j):
            hist_v[pl.ds(j * L, L)] = jnp.zeros((L,), jnp.int32)

    def half_pass(img, cls, base_ch, n_ch):
        def start(ch):
            buf = ch % 2
            cp = pltpu.async_copy(
                p_hbm.at[img, cls, pl.ds((base_ch + ch) * RPC, RPC), :],
                p_v.at[buf], sem_p)
            cl = pltpu.async_copy(
                lab_hbm.at[img, pl.ds((base_ch + ch) * RPC, RPC), :],
                lab_v.at[buf], sem_l)
            return cp, cl

        cur = start(0)
        for ch in range(n_ch):
            cp, cl = cur
            cp.wait()
            cl.wait()
            if ch + 1 < n_ch:
                cur = start(ch + 1)
            buf = ch % 2

            @plsc.parallel_loop(0, RPC * 2, unroll=4)
            def _(i):
                r = i >> 1
                hbase = (i & 1) * (GPR // 2 * L)
                for gg in range(GPR // 2):
                    sl = pl.ds(hbase + gg * L, L)
                    p = p_v[buf, r, sl]
                    lab = lab_v[buf, r, sl]
                    fg = lab == cls
                    kp = (p * float(NB)).astype(jnp.int32)
                    k2 = kp + jnp.where(fg, jnp.int32(0), jnp.int32(NB))
                    plsc.addupdate_scatter(hist_v, [k2], ones)

    def unit_scan(slot):
        @plsc.parallel_loop(0, NB // L, unroll=4, carry=jnp.int32(0))
        def ftot(j, acc):
            return acc + jnp.sum(hist_v[pl.ds(j * L, L)])

        g = ftot.astype(jnp.float32)
        zero = jnp.float32(0.0)

        @plsc.parallel_loop(0, NB // L, unroll=4, carry=(zero, zero, zero))
        def scans(j, carry):
            msuf, fsuf, jsum = carry
            jj = NB // L - 1 - j
            cf = hist_v[pl.ds(jj * L, L)].astype(jnp.float32)
            cb = lax.rev(
                hist_v[pl.ds(2 * NB - L * (jj + 1), L)], (0,)
            ).astype(jnp.float32)
            ca = cf + cb
            cum_a = plsc.cumsum(ca)
            cum_f = plsc.cumsum(cf)
            sa = jnp.sum(ca)
            sf = jnp.sum(cf)
            m = msuf + sa - cum_a + ca
            f = fsuf + sf - cum_f + cf
            union = g + m - f
            inter = g - f
            jac = 1.0 - inter / jnp.maximum(union, 1.0)
            return (msuf + sa, fsuf + sf, jsum + jnp.sum(jac))

        _, _, jsum = scans
        loss = (jsum - 0.5) * jnp.float32(1.0 / NB)
        io = lax.broadcasted_iota(jnp.int32, (L,), 0)
        vec = jnp.where(io == 0, loss, jnp.where(io == 1, g, 0.0))
        res_v[pl.ds(slot * L, L)] = vec

    def unit_body(u, _):
        unit = u * NW + w
        img = unit // C
        cls = unit % C
        zero_hist()
        half_pass(img, cls, 0, NCHU)
        unit_scan(u)
        return 0

    lax.fori_loop(0, 2, unit_body, 0)

    res_v[pl.ds(2 * L, L)] = jnp.zeros((L,), jnp.float32)
    half = NTAIL // NC

    @pl.when(sid < half)
    def _():
        tu = 2 * NW + sid * NC + cid
        zero_hist()
        half_pass(tu // C, tu % C, 0, 4)

    @pl.when((sid >= half) & (sid < 2 * half))
    def _():
        tu = 2 * NW + (sid - half) * NC + cid
        zero_hist()
        half_pass(tu // C, tu % C, 4, 3)
        pltpu.sync_copy(hist_v, shared.at[sid])

    plsc.subcore_barrier()

    @pl.when(sid < half)
    def _():
        pltpu.sync_copy(shared.at[sid + half], merge_v)

        @plsc.parallel_loop(0, 2 * NB // L, unroll=8)
        def _(j):
            sl = pl.ds(j * L, L)
            hist_v[sl] = hist_v[sl] + merge_v[sl]

        unit_scan(2)

    pltpu.sync_copy(res_v, out_hbm.at[w])


_hist_call = pl.kernel(
    _sc_body,
    out_type=jax.ShapeDtypeStruct((NW, 3 * L), jnp.float32),
    mesh=plsc.VectorSubcoreMesh(core_axis_name="c", subcore_axis_name="s"),
    compiler_params=pltpu.CompilerParams(
        needs_layout_passes=False, use_tc_tiling_on_sc=True),
    scratch_types=[
        pltpu.VMEM((2, RPC, W), jnp.float32),
        pltpu.VMEM((2, RPC, W), jnp.int32),
        pltpu.VMEM((2 * NB,), jnp.int32),
        pltpu.VMEM((2 * NB,), jnp.int32),
        pltpu.VMEM((3 * L,), jnp.float32),
        pltpu.VMEM_SHARED((NS, 2 * NB), jnp.int32),
        pltpu.SemaphoreType.DMA,
        pltpu.SemaphoreType.DMA,
    ],
)

def kernel(probas, labels):
    lab = labels.astype(jnp.int32)
    out = _hist_call(probas, lab).reshape(NW, 3, L)
    loss = jnp.concatenate(
        [out[:, 0, 0], out[:, 1, 0], out[:NTAIL, 2, 0]]).reshape(B, C)
    g = jnp.concatenate(
        [out[:, 0, 1], out[:, 1, 1], out[:NTAIL, 2, 1]]).reshape(B, C)
    mask = (g > 0).astype(jnp.float32)
    per_img = jnp.sum(loss * mask, axis=1) / jnp.sum(mask, axis=1)
    return jnp.mean(per_img)


# speedup vs baseline: 1.0239x; 1.0239x over previous
import jax
import jax.numpy as jnp
from jax import lax
from jax.experimental import pallas as pl
from jax.experimental.pallas import tpu as pltpu
from jax.experimental.pallas import tpu_sc as plsc

B, C, H, W = 4, 19, 224, 224
N = H * W
NUNITS = B * C
NC, NS, L = 2, 16, 16
NW = NC * NS
NTAIL = NUNITS - 2 * NW
RPC = 32
CH = RPC * W
GPR = W // L
NCHU = H // RPC
NB = 1024


def _sc_body(p_hbm, lab_hbm, out_hbm, p_v, lab_v, hist_v, merge_v, res_v,
             shared, sem_p, sem_l):
    cid = lax.axis_index("c")
    sid = lax.axis_index("s")
    w = sid * NC + cid
    ones = jnp.ones((L,), jnp.int32)

    def zero_hist():
        @plsc.parallel_loop(0, 2 * NB // L, unroll=8)
        def _(j):
            hist_v[pl.ds(j * L, L)] = jnp.zeros((L,), jnp.int32)

    def half_pass(img, cls, base_ch, n_ch):
        def start(ch):
            buf = ch % 2
            cp = pltpu.async_copy(
                p_hbm.at[img, cls, pl.ds((base_ch + ch) * RPC, RPC), :],
                p_v.at[buf], sem_p)
            cl = pltpu.async_copy(
                lab_hbm.at[img, pl.ds((base_ch + ch) * RPC, RPC), :],
                lab_v.at[buf], sem_l)
            return cp, cl

        cur = start(0)
        for ch in range(n_ch):
            cp, cl = cur
            cp.wait()
            cl.wait()
            if ch + 1 < n_ch:
                cur = start(ch + 1)
            buf = ch % 2

            @plsc.parallel_loop(0, RPC * 2, unroll=2)
            def _(i):
                r = i >> 1
                hbase = (i & 1) * (GPR // 2 * L)
                for gg in range(GPR // 2):
                    sl = pl.ds(hbase + gg * L, L)
                    p = p_v[buf, r, sl]
                    lab = lab_v[buf, r, sl]
                    fg = lab == cls
                    kp = (p * float(NB)).astype(jnp.int32)
                    k2 = kp + jnp.where(fg, jnp.int32(0), jnp.int32(NB))
                    plsc.addupdate_scatter(hist_v, [k2], ones)

    def unit_scan(slot):
        @plsc.parallel_loop(0, NB // L, unroll=4, carry=jnp.int32(0))
        def ftot(j, acc):
            return acc + jnp.sum(hist_v[pl.ds(j * L, L)])

        g = ftot.astype(jnp.float32)
        zero = jnp.float32(0.0)

        @plsc.parallel_loop(0, NB // L, unroll=4, carry=(zero, zero, zero))
        def scans(j, carry):
            msuf, fsuf, jsum = carry
            jj = NB // L - 1 - j
            cf = hist_v[pl.ds(jj * L, L)].astype(jnp.float32)
            cb = lax.rev(
                hist_v[pl.ds(2 * NB - L * (jj + 1), L)], (0,)
            ).astype(jnp.float32)
            ca = cf + cb
            cum_a = plsc.cumsum(ca)
            cum_f = plsc.cumsum(cf)
            sa = jnp.sum(ca)
            sf = jnp.sum(cf)
            m = msuf + sa - cum_a + ca
            f = fsuf + sf - cum_f + cf
            union = g + m - f
            inter = g - f
            jac = 1.0 - inter / jnp.maximum(union, 1.0)
            return (msuf + sa, fsuf + sf, jsum + jnp.sum(jac))

        _, _, jsum = scans
        loss = (jsum - 0.5) * jnp.float32(1.0 / NB)
        io = lax.broadcasted_iota(jnp.int32, (L,), 0)
        vec = jnp.where(io == 0, loss, jnp.where(io == 1, g, 0.0))
        res_v[pl.ds(slot * L, L)] = vec

    def unit_body(u, _):
        unit = u * NW + w
        img = unit // C
        cls = unit % C
        zero_hist()
        half_pass(img, cls, 0, NCHU)
        unit_scan(u)
        return 0

    lax.fori_loop(0, 2, unit_body, 0)

    res_v[pl.ds(2 * L, L)] = jnp.zeros((L,), jnp.float32)
    half = NTAIL // NC

    @pl.when(sid < half)
    def _():
        tu = 2 * NW + sid * NC + cid
        zero_hist()
        half_pass(tu // C, tu % C, 0, 4)

    @pl.when((sid >= half) & (sid < 2 * half))
    def _():
        tu = 2 * NW + (sid - half) * NC + cid
        zero_hist()
        half_pass(tu // C, tu % C, 4, 3)
        pltpu.sync_copy(hist_v, shared.at[sid])

    plsc.subcore_barrier()

    @pl.when(sid < half)
    def _():
        pltpu.sync_copy(shared.at[sid + half], merge_v)

        @plsc.parallel_loop(0, 2 * NB // L, unroll=8)
        def _(j):
            sl = pl.ds(j * L, L)
            hist_v[sl] = hist_v[sl] + merge_v[sl]

        unit_scan(2)

    pltpu.sync_copy(res_v, out_hbm.at[w])


_hist_call = pl.kernel(
    _sc_body,
    out_type=jax.ShapeDtypeStruct((NW, 3 * L), jnp.float32),
    mesh=plsc.VectorSubcoreMesh(core_axis_name="c", subcore_axis_name="s"),
    compiler_params=pltpu.CompilerParams(
        needs_layout_passes=False, use_tc_tiling_on_sc=True),
    scratch_types=[
        pltpu.VMEM((2, RPC, W), jnp.float32),
        pltpu.VMEM((2, RPC, W), jnp.int32),
        pltpu.VMEM((2 * NB,), jnp.int32),
        pltpu.VMEM((2 * NB,), jnp.int32),
        pltpu.VMEM((3 * L,), jnp.float32),
        pltpu.VMEM_SHARED((NS, 2 * NB), jnp.int32),
        pltpu.SemaphoreType.DMA,
        pltpu.SemaphoreType.DMA,
    ],
)

def kernel(probas, labels):
    lab = labels.astype(jnp.int32)
    out = _hist_call(probas, lab).reshape(NW, 3, L)
    loss = jnp.concatenate(
        [out[:, 0, 0], out[:, 1, 0], out[:NTAIL, 2, 0]]).reshape(B, C)
    g = jnp.concatenate(
        [out[:, 0, 1], out[:, 1, 1], out[:NTAIL, 2, 1]]).reshape(B, C)
    mask = (g > 0).astype(jnp.float32)
    per_img = jnp.sum(loss * mask, axis=1) / jnp.sum(mask, axis=1)
    return jnp.mean(per_img)
